# Initial kernel scaffold; baseline (speedup 1.0000x reference)
import jax
import jax.numpy as jnp
from jax.experimental import pallas as pl


def kernel(inputs, token_table, segment_table, position_table):
    raise NotImplementedError("write your pallas kernel here")


# speedup vs baseline: 1.8159x; 1.8159x over previous
import functools

import jax
import jax.numpy as jnp
from jax import lax
from jax.experimental import pallas as pl
from jax.experimental.pallas import tpu as pltpu
from jax.experimental.pallas import tpu_sc as plsc

_L = 16


def _make_sc_kernel(B, S, D, V):
    NC, NS = 2, 16
    NW = NC * NS
    RPW = B // NW
    HALF = S // 2
    CH = D // _L

    mesh = plsc.VectorSubcoreMesh(core_axis_name="c", subcore_axis_name="s")

    full_groups = S // _L
    rem = S - full_groups * _L

    @functools.partial(
        pl.kernel,
        out_type=jax.ShapeDtypeStruct((B, S, D), jnp.float32),
        mesh=mesh,
        scratch_types=[
            pltpu.VMEM((2, HALF), jnp.int32),
            pltpu.VMEM((S,), jnp.int32),
            pltpu.VMEM((S, D), jnp.float32),
            pltpu.VMEM((2 * S, D), jnp.float32),
            pltpu.VMEM((2, D), jnp.float32),
            pltpu.SemaphoreType.DMA,
        ],
        compiler_params=pltpu.CompilerParams(use_tc_tiling_on_sc=False),
    )
    def sc_kernel(idx2_hbm, idxf_hbm, tok_hbm, seg_hbm, pos_hbm, out_hbm,
                  idx_v, idxf_v, rows, comb, seg_v, gsem):
        wid = lax.axis_index("s") * NC + lax.axis_index("c")
        lane = lax.iota(jnp.int32, _L)

        pltpu.sync_copy(pos_hbm.at[pl.ds(0, S)], comb.at[pl.ds(0, S)])
        pltpu.sync_copy(pos_hbm.at[pl.ds(0, S)], comb.at[pl.ds(S, S)])
        pltpu.sync_copy(seg_hbm, seg_v)

        def _comb_body(j, carry):
            for c in range(CH):
                sl = pl.ds(c * _L, _L)
                comb[j, sl] = comb[j, sl] + seg_v[0, sl]
                comb[S + j, sl] = comb[S + j, sl] + seg_v[1, sl]
            return carry
        lax.fori_loop(0, S, _comb_body, 0)

        def _add_group(base, lo):
            src16 = (jnp.minimum(idxf_v[pl.ds(base, _L)], 1) * S
                     + base + lane)
            for l in range(lo, _L):
                src = src16[l]
                j = base + l
                for c in range(CH):
                    sl = pl.ds(c * _L, _L)
                    rows[j, sl] = rows[j, sl] + comb[src, sl]

        def _row_body(i, carry):
            gb = wid * RPW + i
            pltpu.sync_copy(idx2_hbm.at[gb], idx_v)
            pltpu.sync_copy(idxf_hbm.at[pl.ds(gb * S, S)], idxf_v)
            cp0 = pltpu.async_copy(tok_hbm.at[idx_v.at[0]],
                                   rows.at[pl.ds(0, HALF)], gsem)
            cp1 = pltpu.async_copy(tok_hbm.at[idx_v.at[1]],
                                   rows.at[pl.ds(HALF, HALF)], gsem)
            cp0.wait()
            cp1.wait()

            def _grp_body(g, c2):
                _add_group(g * _L, 0)
                return c2
            lax.fori_loop(0, full_groups, _grp_body, 0)
            if rem:
                _add_group(S - _L, _L - rem)

            pltpu.sync_copy(rows, out_hbm.at[gb])
            return carry
        lax.fori_loop(0, RPW, _row_body, 0)

    return sc_kernel


def kernel(inputs, token_table, segment_table, position_table):
    B, S = inputs.shape
    V, D = token_table.shape
    idx = inputs.astype(jnp.int32)
    idx2 = idx.reshape(B, 2, S // 2)
    idxf = idx.reshape(B * S)
    k = _make_sc_kernel(B, S, D, V)
    return k(idx2, idxf, token_table, segment_table, position_table)


# speedup vs baseline: 2.2891x; 1.2605x over previous
import functools

import jax
import jax.numpy as jnp
from jax import lax
from jax.experimental import pallas as pl
from jax.experimental.pallas import tpu as pltpu
from jax.experimental.pallas import tpu_sc as plsc

_L = 16
_IL = 128


def _make_sc_kernel(B, S, D, V):
    NC, NS = 2, 16
    NW = NC * NS
    TOT = B * S
    CPW = TOT // NW
    CHK = 2 * _IL
    NCHK = CPW // CHK
    NB = 4
    LOOKAHEAD = 2
    CH = D // _L

    mesh = plsc.VectorSubcoreMesh(core_axis_name="c", subcore_axis_name="s")

    scratch = (
        [pltpu.VMEM((2, _IL), jnp.int32) for _ in range(NB)]
        + [pltpu.VMEM((CHK, D), jnp.float32) for _ in range(NB)]
        + [pltpu.VMEM((2 * S, D), jnp.float32),
           pltpu.VMEM((2, D), jnp.float32)]
        + [pltpu.SemaphoreType.DMA for _ in range(2 * NB)]
    )

    @functools.partial(
        pl.kernel,
        out_type=jax.ShapeDtypeStruct((TOT, D), jnp.float32),
        mesh=mesh,
        scratch_types=scratch,
        compiler_params=pltpu.CompilerParams(use_tc_tiling_on_sc=False),
    )
    def sc_kernel(idx2d_hbm, tok_hbm, seg_hbm, pos_hbm, out_hbm, *refs):
        idxs = refs[0:NB]
        rows = refs[NB:2 * NB]
        comb = refs[2 * NB]
        seg_v = refs[2 * NB + 1]
        gsem = refs[2 * NB + 2:2 * NB + 2 + NB]
        osem = refs[2 * NB + 2 + NB:]

        wid = lax.axis_index("s") * NC + lax.axis_index("c")
        wbase = wid * CPW
        wrow = wid * (CPW // _IL)
        lane = lax.iota(jnp.int32, _L)

        pltpu.sync_copy(pos_hbm.at[pl.ds(0, S)], comb.at[pl.ds(0, S)])
        pltpu.sync_copy(pos_hbm.at[pl.ds(0, S)], comb.at[pl.ds(S, S)])
        pltpu.sync_copy(seg_hbm, seg_v)

        def _comb_body(j, carry):
            for ci in range(CH):
                sl = pl.ds(ci * _L, _L)
                comb[j, sl] = comb[j, sl] + seg_v[0, sl]
                comb[S + j, sl] = comb[S + j, sl] + seg_v[1, sl]
            return carry
        lax.fori_loop(0, S, _comb_body, 0)

        def _issue_gather(c, b):
            pltpu.sync_copy(idx2d_hbm.at[pl.ds(wrow + c * 2, 2)], idxs[b])
            pltpu.async_copy(tok_hbm.at[idxs[b].at[0]],
                             rows[b].at[pl.ds(0, _IL)], gsem[b])
            pltpu.async_copy(tok_hbm.at[idxs[b].at[1]],
                             rows[b].at[pl.ds(_IL, _IL)], gsem[b])

        def _wait_gather(b):
            for k in range(2):
                pltpu.make_async_copy(tok_hbm.at[idxs[b].at[k]],
                                      rows[b].at[pl.ds(k * _IL, _IL)],
                                      gsem[b]).wait()

        def _drain_out(b):
            pltpu.make_async_copy(rows[b], out_hbm.at[pl.ds(0, CHK)],
                                  osem[b]).wait()

        for p in range(LOOKAHEAD):
            _issue_gather(p, p)

        def _chunk_body(it, carry):
            for b in range(NB):
                c = it * NB + b
                _wait_gather(b)

                for k in range(2):
                    def _grp(g, c2):
                        off = k * _IL + g * _L
                        t16 = jnp.minimum(idxs[b][k, pl.ds(g * _L, _L)], 1)
                        pos16 = lax.rem(wbase + c * CHK + off + lane,
                                        jnp.int32(S))
                        src16 = t16 * S + pos16
                        for l in range(_L):
                            src = src16[l]
                            il = off + l
                            for ci in range(CH):
                                sl = pl.ds(ci * _L, _L)
                                plsc.addupdate(rows[b].at[il, sl],
                                               comb[src, sl])
                        return c2
                    lax.fori_loop(0, _IL // _L, _grp, 0)

                pltpu.async_copy(rows[b], out_hbm.at[pl.ds(wbase + c * CHK,
                                                           CHK)], osem[b])

                nc = c + LOOKAHEAD
                nb2 = (b + LOOKAHEAD) % NB

                @pl.when(nc < NCHK)
                def _ahead():
                    @pl.when(c >= LOOKAHEAD)
                    def _drain():
                        _drain_out(nb2)
                    _issue_gather(nc, nb2)
            return carry
        lax.fori_loop(0, NCHK // NB, _chunk_body, 0)

        for b in range(NB):
            _drain_out(b)

    return sc_kernel


def kernel(inputs, token_table, segment_table, position_table):
    B, S = inputs.shape
    V, D = token_table.shape
    idx2d = inputs.astype(jnp.int32).reshape(B * S // _IL, _IL)
    k = _make_sc_kernel(B, S, D, V)
    out = k(idx2d, token_table, segment_table, position_table)
    return out.reshape(B, S, D)
